# Initial kernel scaffold; baseline (speedup 1.0000x reference)
import jax
import jax.numpy as jnp
from jax.experimental import pallas as pl


def kernel(l_embedding, c_embedding, pos_edge_index, neg_edge_index, lm_W1, lm_b1, lm_W2, lm_b2, cm_W1, cm_b1, cm_W2, cm_b2, lu_W1, lu_b1, lu_W2, lu_b2, cu_W1, cu_b1, cu_W2, cu_b2):
    raise NotImplementedError("write your pallas kernel here")


# speedup vs baseline: 3.9827x; 3.9827x over previous
import functools

import jax
import jax.numpy as jnp
from jax import lax
from jax.experimental import pallas as pl
from jax.experimental.pallas import tpu as pltpu
from jax.experimental.pallas import tpu_sc as plsc

HID = 128
NHALF = 5000
NEDGE = 160000
NROUND = 4

NSC = 2
NSUB = 16
KCH = 80
EPT = NEDGE // NSUB
NCHUNK = EPT // KCH

ZROWS = 200
NZCH = NHALF // ZROWS

BLK = 1000
GRID = NHALF // BLK


def _sc_edge_sums(pm, nm, cm, ps3, pd3, ns3, nd3, zrows):
    mesh = plsc.VectorSubcoreMesh(core_axis_name="c", subcore_axis_name="s",
                                  num_cores=NSC, num_subcores=NSUB)
    out_type = [jax.ShapeDtypeStruct((NHALF, HID), jnp.float32)] * 3
    scratch = [
        pltpu.VMEM_SHARED((NHALF, HID), jnp.float32),
        pltpu.VMEM_SHARED((NHALF, HID), jnp.float32),
        pltpu.VMEM((NCHUNK, KCH), jnp.int32),
        pltpu.VMEM((NCHUNK, KCH), jnp.int32),
        pltpu.VMEM((KCH, HID), jnp.float32),
        pltpu.SemaphoreType.DMA,
        pltpu.SemaphoreType.DMA,
    ]

    @functools.partial(pl.kernel, out_type=out_type, mesh=mesh,
                       scratch_types=scratch)
    def k(pm_h, nm_h, cm_h, ps_h, pd_h, ns_h, nd_h, z_h,
          l2c_h, pc2l_h, nc2l_h,
          s1, s2, gidx, sidx, rows, gsem, ssem):
        sid = lax.axis_index("s")
        cid = lax.axis_index("c")

        def zero(dst_sp):
            for j in range(-(-NZCH // NSUB)):
                ch = sid + j * NSUB

                @pl.when(ch < NZCH)
                def _():
                    sl = pl.ds(pl.multiple_of(ch * ZROWS, 8), ZROWS)
                    pltpu.sync_copy(z_h, dst_sp.at[sl])

        def stage_out(src_sp, dst_hbm):
            for j in range(-(-NZCH // NSUB)):
                ch = sid + j * NSUB

                @pl.when(ch < NZCH)
                def _():
                    sl = pl.ds(pl.multiple_of(ch * ZROWS, 8), ZROWS)
                    pltpu.sync_copy(src_sp.at[sl], dst_hbm.at[sl])

        def edge_pass(gih, tab, sih, acc):
            pltpu.sync_copy(gih.at[sid], gidx)
            pltpu.sync_copy(sih.at[sid], sidx)

            def body(i, carry):
                pltpu.async_copy(tab.at[gidx.at[i]], rows, gsem).wait()
                pltpu.async_copy(rows, acc.at[sidx.at[i]], ssem,
                                 add=True).wait()
                return carry

            lax.fori_loop(0, NCHUNK, body, 0)

        @pl.when(cid == 0)
        def _():
            zero(s1)
            plsc.subcore_barrier()
            edge_pass(ps_h, pm_h, pd_h, s1)
            edge_pass(ns_h, nm_h, nd_h, s1)
            plsc.subcore_barrier()
            stage_out(s1, l2c_h)

        @pl.when(cid == 1)
        def _():
            zero(s1)
            zero(s2)
            plsc.subcore_barrier()
            edge_pass(pd_h, cm_h, ps_h, s1)
            edge_pass(nd_h, cm_h, ns_h, s2)
            plsc.subcore_barrier()
            stage_out(s1, pc2l_h)
            stage_out(s2, nc2l_h)

    return k(pm, nm, cm, ps3, pd3, ns3, nd3, zrows)


def _mlp(x, w1, b1, w2, b2):
    h = jnp.maximum(jnp.dot(x, w1, preferred_element_type=jnp.float32) + b1,
                    0.0)
    return jnp.dot(h, w2, preferred_element_type=jnp.float32) + b2


def _bspec():
    return pl.BlockSpec((BLK, HID), lambda i: (i, 0))


def _wspec(shape):
    return pl.BlockSpec(shape, lambda i: (0, 0))


def _tc_msgs(lp, ln, c, lmW1, lmb1, lmW2, lmb2, cmW1, cmb1, cmW2, cmb2):
    def body(lp_r, ln_r, c_r, w1, b1, w2, b2, v1, a1, v2, a2,
             pm_r, nm_r, cm_r):
        pm_r[...] = _mlp(lp_r[...], w1[...], b1[...], w2[...], b2[...])
        nm_r[...] = _mlp(ln_r[...], w1[...], b1[...], w2[...], b2[...])
        cm_r[...] = _mlp(c_r[...], v1[...], a1[...], v2[...], a2[...])

    w = _wspec((HID, HID))
    b = _wspec((1, HID))
    return pl.pallas_call(
        body,
        grid=(GRID,),
        in_specs=[_bspec()] * 3 + [w, b, w, b, w, b, w, b],
        out_specs=[_bspec()] * 3,
        out_shape=[jax.ShapeDtypeStruct((NHALF, HID), jnp.float32)] * 3,
    )(lp, ln, c, lmW1, lmb1, lmW2, lmb2, cmW1, cmb1, cmW2, cmb2)


def _tc_update(l2c, pc2l, nc2l, lp, ln, c, wd, with_msgs):
    def body(*refs):
        (l2c_r, pc2l_r, nc2l_r, lp_r, ln_r, c_r,
         cu1a, cu1b, cub1, cu2, cub2,
         lu1a, lu1b, lu1c, lub1, lu2, lub2) = refs[:17]
        if with_msgs:
            (lm1, lmb1, lm2, lmb2, cm1, cmb1, cm2, cmb2) = refs[17:25]
            (lp_o, ln_o, c_o, pm_o, nm_o, cm_o) = refs[25:]
        else:
            (lp_o, ln_o, c_o) = refs[17:]

        dot = lambda x, w: jnp.dot(x[...], w[...],
                                   preferred_element_type=jnp.float32)
        hc = jnp.maximum(dot(l2c_r, cu1a) + dot(c_r, cu1b) + cub1[...], 0.0)
        c_new = jnp.dot(hc, cu2[...],
                        preferred_element_type=jnp.float32) + cub2[...]
        hp = jnp.maximum(dot(pc2l_r, lu1a) + dot(lp_r, lu1b)
                         + dot(ln_r, lu1c) + lub1[...], 0.0)
        lp_new = jnp.dot(hp, lu2[...],
                         preferred_element_type=jnp.float32) + lub2[...]
        hn = jnp.maximum(dot(nc2l_r, lu1a) + dot(ln_r, lu1b)
                         + dot(lp_r, lu1c) + lub1[...], 0.0)
        ln_new = jnp.dot(hn, lu2[...],
                         preferred_element_type=jnp.float32) + lub2[...]
        lp_o[...] = lp_new
        ln_o[...] = ln_new
        c_o[...] = c_new
        if with_msgs:
            pm_o[...] = _mlp(lp_new, lm1[...], lmb1[...], lm2[...], lmb2[...])
            nm_o[...] = _mlp(ln_new, lm1[...], lmb1[...], lm2[...], lmb2[...])
            cm_o[...] = _mlp(c_new, cm1[...], cmb1[...], cm2[...], cmb2[...])

    w = _wspec((HID, HID))
    b = _wspec((1, HID))
    n_out = 6 if with_msgs else 3
    wspecs = [w, w, b, w, b, w, w, w, b, w, b]
    if with_msgs:
        wspecs += [w, b, w, b, w, b, w, b]
    return pl.pallas_call(
        body,
        grid=(GRID,),
        in_specs=[_bspec()] * 6 + wspecs,
        out_specs=[_bspec()] * n_out,
        out_shape=[jax.ShapeDtypeStruct((NHALF, HID), jnp.float32)] * n_out,
    )(l2c, pc2l, nc2l, lp, ln, c, *wd)


def kernel(l_embedding, c_embedding, pos_edge_index, neg_edge_index,
           lm_W1, lm_b1, lm_W2, lm_b2,
           cm_W1, cm_b1, cm_W2, cm_b2,
           lu_W1, lu_b1, lu_W2, lu_b2,
           cu_W1, cu_b1, cu_W2, cu_b2):
    lp = l_embedding[:NHALF]
    ln = l_embedding[NHALF:]
    c = c_embedding

    ps3 = pos_edge_index[0].reshape(NSUB, NCHUNK, KCH).astype(jnp.int32)
    pd3 = pos_edge_index[1].reshape(NSUB, NCHUNK, KCH).astype(jnp.int32)
    ns3 = neg_edge_index[0].reshape(NSUB, NCHUNK, KCH).astype(jnp.int32)
    nd3 = neg_edge_index[1].reshape(NSUB, NCHUNK, KCH).astype(jnp.int32)
    zrows = jnp.zeros((ZROWS, HID), jnp.float32)

    lmb1 = lm_b1.reshape(1, HID)
    lmb2 = lm_b2.reshape(1, HID)
    cmb1 = cm_b1.reshape(1, HID)
    cmb2 = cm_b2.reshape(1, HID)
    lub1 = lu_b1.reshape(1, HID)
    lub2 = lu_b2.reshape(1, HID)
    cub1 = cu_b1.reshape(1, HID)
    cub2 = cu_b2.reshape(1, HID)
    cu1a, cu1b = cu_W1[:HID], cu_W1[HID:]
    lu1a, lu1b, lu1c = lu_W1[:HID], lu_W1[HID:2 * HID], lu_W1[2 * HID:]

    upd_w = (cu1a, cu1b, cub1, cu_W2, cub2,
             lu1a, lu1b, lu1c, lub1, lu_W2, lub2)
    msg_w = (lm_W1, lmb1, lm_W2, lmb2, cm_W1, cmb1, cm_W2, cmb2)

    pm, nm, cmsg = _tc_msgs(lp, ln, c, *msg_w)
    for r in range(NROUND):
        l2c, pc2l, nc2l = _sc_edge_sums(pm, nm, cmsg, ps3, pd3, ns3, nd3,
                                        zrows)
        if r < NROUND - 1:
            lp, ln, c, pm, nm, cmsg = _tc_update(
                l2c, pc2l, nc2l, lp, ln, c, upd_w + msg_w, True)
        else:
            lp, ln, c = _tc_update(
                l2c, pc2l, nc2l, lp, ln, c, upd_w, False)

    return (jnp.concatenate([lp, ln], axis=0), c)


# speedup vs baseline: 5.2625x; 1.3214x over previous
import functools

import jax
import jax.numpy as jnp
from jax import lax
from jax.experimental import pallas as pl
from jax.experimental.pallas import tpu as pltpu
from jax.experimental.pallas import tpu_sc as plsc

HID = 128
NHALF = 5000
NEDGE = 160000
NROUND = 4

NSC = 2
NSUB = 16
KCH = 80
EPT = NEDGE // NSUB
NCHUNK = EPT // KCH

ZROWS = 200
NZCH = NHALF // ZROWS

BLK = 1000
GRID = NHALF // BLK


def _sc_edge_sums(pm, nm, cm, idx_arrays, zrows):
    mesh = plsc.VectorSubcoreMesh(core_axis_name="c", subcore_axis_name="s",
                                  num_cores=NSC, num_subcores=NSUB)
    out_type = [jax.ShapeDtypeStruct((NHALF, HID), jnp.float32)] * 3
    scratch = [
        pltpu.VMEM_SHARED((NHALF, HID), jnp.float32),
        pltpu.VMEM_SHARED((NHALF, HID), jnp.float32),
        pltpu.VMEM((EPT,), jnp.int32),
        pltpu.VMEM((1, KCH), jnp.int32),
        pltpu.VMEM((1, KCH), jnp.int32),
        pltpu.VMEM((KCH, HID), jnp.float32),
        pltpu.VMEM((KCH, HID), jnp.float32),
        pltpu.SemaphoreType.DMA,
        pltpu.SemaphoreType.DMA,
        pltpu.SemaphoreType.DMA,
        pltpu.SemaphoreType.DMA,
        pltpu.SemaphoreType.DMA,
        pltpu.SemaphoreType.DMA,
    ]

    @functools.partial(pl.kernel, out_type=out_type, mesh=mesh,
                       scratch_types=scratch)
    def k(pm_h, nm_h, cm_h, psg_h, pss_h, pdg_h, pds_h,
          nsg_h, nss_h, ndg_h, nds_h, z_h,
          l2c_h, pc2l_h, nc2l_h,
          s1, s2, gidx, sbufA, sbufB, rowsA, rowsB,
          gsA, gsB, ssA, ssB, siA, siB):
        sid = lax.axis_index("s")
        cid = lax.axis_index("c")

        def zero(dst_sp):
            for j in range(-(-NZCH // NSUB)):
                ch = sid + j * NSUB

                @pl.when(ch < NZCH)
                def _():
                    sl = pl.ds(pl.multiple_of(ch * ZROWS, 8), ZROWS)
                    pltpu.sync_copy(z_h, dst_sp.at[sl])

        def stage_out(src_sp, dst_hbm):
            for j in range(-(-NZCH // NSUB)):
                ch = sid + j * NSUB

                @pl.when(ch < NZCH)
                def _():
                    sl = pl.ds(pl.multiple_of(ch * ZROWS, 8), ZROWS)
                    pltpu.sync_copy(src_sp.at[sl], dst_hbm.at[sl])

        def edge_pass(gih, tab, sih, acc):
            nloop = NCHUNK // 2

            def gath(c, buf, sem):
                idxs = gidx.at[pl.ds(pl.multiple_of(c * KCH, 8), KCH)]
                return pltpu.async_copy(tab.at[idxs], buf, sem)

            def gath_wait(c, buf, sem):
                idxs = gidx.at[pl.ds(pl.multiple_of(c * KCH, 8), KCH)]
                pltpu.make_async_copy(tab.at[idxs], buf, sem).wait()

            def sload(c, sb, sem):
                pltpu.async_copy(sih.at[sid, c], sb, sem)

            def swait(c, sb, sem):
                pltpu.make_async_copy(sih.at[sid, c], sb, sem).wait()

            def scat(c, buf, sb, sem):
                return pltpu.async_copy(buf, acc.at[sb.at[0]], sem,
                                        add=True)

            def scat_wait(c, buf, sb, sem):
                pltpu.make_async_copy(buf, acc.at[sb.at[0]], sem).wait()

            pltpu.sync_copy(gih.at[sid], gidx)
            gath(0, rowsA, gsA)
            sload(0, sbufA, siA)
            gath(1, rowsB, gsB)
            sload(1, sbufB, siB)

            def body(i, carry):
                ca = 2 * i
                cb = ca + 1
                gath_wait(ca, rowsA, gsA)
                swait(ca, sbufA, siA)
                scat(ca, rowsA, sbufA, ssA)
                gath_wait(cb, rowsB, gsB)
                swait(cb, sbufB, siB)
                scat(cb, rowsB, sbufB, ssB)
                scat_wait(ca, rowsA, sbufA, ssA)
                gath(ca + 2, rowsA, gsA)
                sload(ca + 2, sbufA, siA)

                @pl.when(i < nloop - 1)
                def _():
                    scat_wait(cb, rowsB, sbufB, ssB)
                    gath(cb + 2, rowsB, gsB)
                    sload(cb + 2, sbufB, siB)

                return carry

            lax.fori_loop(0, nloop, body, 0)
            scat_wait(NCHUNK - 2, rowsB, sbufB, ssB)
            gath_wait(NCHUNK - 1, rowsA, gsA)
            swait(NCHUNK - 1, sbufA, siA)
            scat(NCHUNK - 1, rowsA, sbufA, ssA)
            scat_wait(NCHUNK - 1, rowsA, sbufA, ssA)

        @pl.when(cid == 0)
        def _():
            zero(s1)
            plsc.subcore_barrier()
            edge_pass(psg_h, pm_h, pds_h, s1)
            edge_pass(nsg_h, nm_h, nds_h, s1)
            plsc.subcore_barrier()
            stage_out(s1, l2c_h)

        @pl.when(cid == 1)
        def _():
            zero(s1)
            zero(s2)
            plsc.subcore_barrier()
            edge_pass(pdg_h, cm_h, pss_h, s1)
            edge_pass(ndg_h, cm_h, nss_h, s2)
            plsc.subcore_barrier()
            stage_out(s1, pc2l_h)
            stage_out(s2, nc2l_h)

    return k(pm, nm, cm, *idx_arrays, zrows)


def _mlp(x, w1, b1, w2, b2):
    h = jnp.maximum(jnp.dot(x, w1, preferred_element_type=jnp.float32) + b1,
                    0.0)
    return jnp.dot(h, w2, preferred_element_type=jnp.float32) + b2


def _bspec():
    return pl.BlockSpec((BLK, HID), lambda i: (i, 0))


def _wspec(shape):
    return pl.BlockSpec(shape, lambda i: (0, 0))


def _tc_msgs(lp, ln, c, lmW1, lmb1, lmW2, lmb2, cmW1, cmb1, cmW2, cmb2):
    def body(lp_r, ln_r, c_r, w1, b1, w2, b2, v1, a1, v2, a2,
             pm_r, nm_r, cm_r):
        pm_r[...] = _mlp(lp_r[...], w1[...], b1[...], w2[...], b2[...])
        nm_r[...] = _mlp(ln_r[...], w1[...], b1[...], w2[...], b2[...])
        cm_r[...] = _mlp(c_r[...], v1[...], a1[...], v2[...], a2[...])

    w = _wspec((HID, HID))
    b = _wspec((1, HID))
    return pl.pallas_call(
        body,
        grid=(GRID,),
        in_specs=[_bspec()] * 3 + [w, b, w, b, w, b, w, b],
        out_specs=[_bspec()] * 3,
        out_shape=[jax.ShapeDtypeStruct((NHALF, HID), jnp.float32)] * 3,
    )(lp, ln, c, lmW1, lmb1, lmW2, lmb2, cmW1, cmb1, cmW2, cmb2)


def _tc_update(l2c, pc2l, nc2l, lp, ln, c, wd, with_msgs):
    def body(*refs):
        (l2c_r, pc2l_r, nc2l_r, lp_r, ln_r, c_r,
         cu1a, cu1b, cub1, cu2, cub2,
         lu1a, lu1b, lu1c, lub1, lu2, lub2) = refs[:17]
        if with_msgs:
            (lm1, lmb1, lm2, lmb2, cm1, cmb1, cm2, cmb2) = refs[17:25]
            (lp_o, ln_o, c_o, pm_o, nm_o, cm_o) = refs[25:]
        else:
            (lp_o, ln_o, c_o) = refs[17:]

        dot = lambda x, w: jnp.dot(x[...], w[...],
                                   preferred_element_type=jnp.float32)
        hc = jnp.maximum(dot(l2c_r, cu1a) + dot(c_r, cu1b) + cub1[...], 0.0)
        c_new = jnp.dot(hc, cu2[...],
                        preferred_element_type=jnp.float32) + cub2[...]
        hp = jnp.maximum(dot(pc2l_r, lu1a) + dot(lp_r, lu1b)
                         + dot(ln_r, lu1c) + lub1[...], 0.0)
        lp_new = jnp.dot(hp, lu2[...],
                         preferred_element_type=jnp.float32) + lub2[...]
        hn = jnp.maximum(dot(nc2l_r, lu1a) + dot(ln_r, lu1b)
                         + dot(lp_r, lu1c) + lub1[...], 0.0)
        ln_new = jnp.dot(hn, lu2[...],
                         preferred_element_type=jnp.float32) + lub2[...]
        lp_o[...] = lp_new
        ln_o[...] = ln_new
        c_o[...] = c_new
        if with_msgs:
            pm_o[...] = _mlp(lp_new, lm1[...], lmb1[...], lm2[...], lmb2[...])
            nm_o[...] = _mlp(ln_new, lm1[...], lmb1[...], lm2[...], lmb2[...])
            cm_o[...] = _mlp(c_new, cm1[...], cmb1[...], cm2[...], cmb2[...])

    w = _wspec((HID, HID))
    b = _wspec((1, HID))
    n_out = 6 if with_msgs else 3
    wspecs = [w, w, b, w, b, w, w, w, b, w, b]
    if with_msgs:
        wspecs += [w, b, w, b, w, b, w, b]
    return pl.pallas_call(
        body,
        grid=(GRID,),
        in_specs=[_bspec()] * 6 + wspecs,
        out_specs=[_bspec()] * n_out,
        out_shape=[jax.ShapeDtypeStruct((NHALF, HID), jnp.float32)] * n_out,
    )(l2c, pc2l, nc2l, lp, ln, c, *wd)


def kernel(l_embedding, c_embedding, pos_edge_index, neg_edge_index,
           lm_W1, lm_b1, lm_W2, lm_b2,
           cm_W1, cm_b1, cm_W2, cm_b2,
           lu_W1, lu_b1, lu_W2, lu_b2,
           cu_W1, cu_b1, cu_W2, cu_b2):
    lp = l_embedding[:NHALF]
    ln = l_embedding[NHALF:]
    c = c_embedding

    def glay(x):
        return x.reshape(NSUB, EPT).astype(jnp.int32)

    def slay(x):
        return x.reshape(NSUB, NCHUNK, 1, KCH).astype(jnp.int32)

    idx_arrays = []
    for arr in (pos_edge_index[0], pos_edge_index[1],
                neg_edge_index[0], neg_edge_index[1]):
        idx_arrays += [glay(arr), slay(arr)]
    zrows = jnp.zeros((ZROWS, HID), jnp.float32)

    lmb1 = lm_b1.reshape(1, HID)
    lmb2 = lm_b2.reshape(1, HID)
    cmb1 = cm_b1.reshape(1, HID)
    cmb2 = cm_b2.reshape(1, HID)
    lub1 = lu_b1.reshape(1, HID)
    lub2 = lu_b2.reshape(1, HID)
    cub1 = cu_b1.reshape(1, HID)
    cub2 = cu_b2.reshape(1, HID)
    cu1a, cu1b = cu_W1[:HID], cu_W1[HID:]
    lu1a, lu1b, lu1c = lu_W1[:HID], lu_W1[HID:2 * HID], lu_W1[2 * HID:]

    upd_w = (cu1a, cu1b, cub1, cu_W2, cub2,
             lu1a, lu1b, lu1c, lub1, lu_W2, lub2)
    msg_w = (lm_W1, lmb1, lm_W2, lmb2, cm_W1, cmb1, cm_W2, cmb2)

    pm, nm, cmsg = _tc_msgs(lp, ln, c, *msg_w)
    for r in range(NROUND):
        l2c, pc2l, nc2l = _sc_edge_sums(pm, nm, cmsg, idx_arrays, zrows)
        if r < NROUND - 1:
            lp, ln, c, pm, nm, cmsg = _tc_update(
                l2c, pc2l, nc2l, lp, ln, c, upd_w + msg_w, True)
        else:
            lp, ln, c = _tc_update(
                l2c, pc2l, nc2l, lp, ln, c, upd_w, False)

    return (jnp.concatenate([lp, ln], axis=0), c)


# speedup vs baseline: 6.5441x; 1.2435x over previous
import functools

import jax
import jax.numpy as jnp
from jax import lax
from jax.experimental import pallas as pl
from jax.experimental.pallas import tpu as pltpu
from jax.experimental.pallas import tpu_sc as plsc

HID = 128
NHALF = 5000
NEDGE = 160000
NROUND = 4

NSC = 2
NSUB = 16
KCH = 40
EPT = NEDGE // NSUB
NCHUNK = EPT // KCH
DEPTH = 4
ZROWS = 200
NZCH = NHALF // ZROWS

BLK = 1000
GRID = NHALF // BLK


def _sc_edge_sums(pm, nm, cm, idx_arrays, zrows):
    mesh = plsc.VectorSubcoreMesh(core_axis_name="c", subcore_axis_name="s",
                                  num_cores=NSC, num_subcores=NSUB)
    out_type = [jax.ShapeDtypeStruct((NHALF, HID), jnp.float32)] * 3
    scratch = (
        [pltpu.VMEM_SHARED((NHALF, HID), jnp.float32)] * 2
        + [pltpu.VMEM((EPT,), jnp.int32)]
        + [pltpu.VMEM((1, KCH), jnp.int32)] * DEPTH
        + [pltpu.VMEM((KCH, HID), jnp.float32)] * DEPTH
        + [pltpu.SemaphoreType.DMA] * (3 * DEPTH)
    )

    @functools.partial(pl.kernel, out_type=out_type, mesh=mesh,
                       scratch_types=scratch)
    def k(pm_h, nm_h, cm_h, psg_h, pss_h, pdg_h, pds_h,
          nsg_h, nss_h, ndg_h, nds_h, z_h,
          l2c_h, pc2l_h, nc2l_h, s1, s2, gidx, *bufs):
        sbuf = bufs[:DEPTH]
        rows = bufs[DEPTH:2 * DEPTH]
        gs = bufs[2 * DEPTH:3 * DEPTH]
        ss = bufs[3 * DEPTH:4 * DEPTH]
        si = bufs[4 * DEPTH:5 * DEPTH]
        sid = lax.axis_index("s")
        cid = lax.axis_index("c")

        def zero(dst_sp):
            for j in range(-(-NZCH // NSUB)):
                ch = sid + j * NSUB

                @pl.when(ch < NZCH)
                def _():
                    sl = pl.ds(pl.multiple_of(ch * ZROWS, 8), ZROWS)
                    pltpu.sync_copy(z_h, dst_sp.at[sl])

        def stage_out(src_sp, dst_hbm):
            for j in range(-(-NZCH // NSUB)):
                ch = sid + j * NSUB

                @pl.when(ch < NZCH)
                def _():
                    sl = pl.ds(pl.multiple_of(ch * ZROWS, 8), ZROWS)
                    pltpu.sync_copy(src_sp.at[sl], dst_hbm.at[sl])

        def edge_pass(gih, tab, sih, acc):
            def gslice(c):
                return gidx.at[pl.ds(pl.multiple_of(c * KCH, 8), KCH)]

            def gath(c, j):
                pltpu.async_copy(tab.at[gslice(c)], rows[j], gs[j])

            def gath_wait(c, j):
                pltpu.make_async_copy(tab.at[gslice(c)], rows[j],
                                      gs[j]).wait()

            def sload(c, j):
                pltpu.async_copy(sih.at[sid, c], sbuf[j], si[j])

            def swait(c, j):
                pltpu.make_async_copy(sih.at[sid, c], sbuf[j],
                                      si[j]).wait()

            def scat(c, j):
                pltpu.async_copy(rows[j], acc.at[sbuf[j].at[0]], ss[j],
                                 add=True)

            def scat_wait(c, j):
                pltpu.make_async_copy(rows[j], acc.at[sbuf[j].at[0]],
                                      ss[j]).wait()

            pltpu.sync_copy(gih.at[sid], gidx)
            for j in range(DEPTH):
                sload(j, j)
                gath(j, j)

            nbody = (NCHUNK - DEPTH) // DEPTH

            def body(i, carry):
                for j in range(DEPTH):
                    c = DEPTH * i + j
                    gath_wait(c, j)
                    swait(c, j)
                    scat(c, j)
                for j in range(DEPTH):
                    c = DEPTH * i + j
                    scat_wait(c, j)
                    gath(c + DEPTH, j)
                    sload(c + DEPTH, j)
                return carry

            lax.fori_loop(0, nbody, body, 0)
            base = nbody * DEPTH
            for c in range(base, base + DEPTH):
                j = c % DEPTH
                gath_wait(c, j)
                swait(c, j)
                scat(c, j)
                scat_wait(c, j)
            for c in range(base + DEPTH, NCHUNK):
                j = c % DEPTH
                gath(c, j)
                sload(c, j)
                gath_wait(c, j)
                swait(c, j)
                scat(c, j)
                scat_wait(c, j)

        @pl.when(cid == 0)
        def _():
            zero(s1)
            plsc.subcore_barrier()
            edge_pass(psg_h, pm_h, pds_h, s1)
            edge_pass(nsg_h, nm_h, nds_h, s1)
            plsc.subcore_barrier()
            stage_out(s1, l2c_h)

        @pl.when(cid == 1)
        def _():
            zero(s1)
            zero(s2)
            plsc.subcore_barrier()
            edge_pass(pdg_h, cm_h, pss_h, s1)
            edge_pass(ndg_h, cm_h, nss_h, s2)
            plsc.subcore_barrier()
            stage_out(s1, pc2l_h)
            stage_out(s2, nc2l_h)

    return k(pm, nm, cm, *idx_arrays, zrows)


def _mlp(x, w1, b1, w2, b2):
    h = jnp.maximum(jnp.dot(x, w1, preferred_element_type=jnp.float32) + b1,
                    0.0)
    return jnp.dot(h, w2, preferred_element_type=jnp.float32) + b2


def _bspec():
    return pl.BlockSpec((BLK, HID), lambda i: (i, 0))


def _wspec(shape):
    return pl.BlockSpec(shape, lambda i: (0, 0))


def _tc_msgs(lp, ln, c, lmW1, lmb1, lmW2, lmb2, cmW1, cmb1, cmW2, cmb2):
    def body(lp_r, ln_r, c_r, w1, b1, w2, b2, v1, a1, v2, a2,
             pm_r, nm_r, cm_r):
        pm_r[...] = _mlp(lp_r[...], w1[...], b1[...], w2[...], b2[...])
        nm_r[...] = _mlp(ln_r[...], w1[...], b1[...], w2[...], b2[...])
        cm_r[...] = _mlp(c_r[...], v1[...], a1[...], v2[...], a2[...])

    w = _wspec((HID, HID))
    b = _wspec((1, HID))
    return pl.pallas_call(
        body,
        grid=(GRID,),
        in_specs=[_bspec()] * 3 + [w, b, w, b, w, b, w, b],
        out_specs=[_bspec()] * 3,
        out_shape=[jax.ShapeDtypeStruct((NHALF, HID), jnp.float32)] * 3,
    )(lp, ln, c, lmW1, lmb1, lmW2, lmb2, cmW1, cmb1, cmW2, cmb2)


def _tc_update(l2c, pc2l, nc2l, lp, ln, c, wd, with_msgs):
    def body(*refs):
        (l2c_r, pc2l_r, nc2l_r, lp_r, ln_r, c_r,
         cu1a, cu1b, cub1, cu2, cub2,
         lu1a, lu1b, lu1c, lub1, lu2, lub2) = refs[:17]
        if with_msgs:
            (lm1, lmb1, lm2, lmb2, cm1, cmb1, cm2, cmb2) = refs[17:25]
            (lp_o, ln_o, c_o, pm_o, nm_o, cm_o) = refs[25:]
        else:
            (lp_o, ln_o, c_o) = refs[17:]

        dot = lambda x, w: jnp.dot(x[...], w[...],
                                   preferred_element_type=jnp.float32)
        hc = jnp.maximum(dot(l2c_r, cu1a) + dot(c_r, cu1b) + cub1[...], 0.0)
        c_new = jnp.dot(hc, cu2[...],
                        preferred_element_type=jnp.float32) + cub2[...]
        hp = jnp.maximum(dot(pc2l_r, lu1a) + dot(lp_r, lu1b)
                         + dot(ln_r, lu1c) + lub1[...], 0.0)
        lp_new = jnp.dot(hp, lu2[...],
                         preferred_element_type=jnp.float32) + lub2[...]
        hn = jnp.maximum(dot(nc2l_r, lu1a) + dot(ln_r, lu1b)
                         + dot(lp_r, lu1c) + lub1[...], 0.0)
        ln_new = jnp.dot(hn, lu2[...],
                         preferred_element_type=jnp.float32) + lub2[...]
        lp_o[...] = lp_new
        ln_o[...] = ln_new
        c_o[...] = c_new
        if with_msgs:
            pm_o[...] = _mlp(lp_new, lm1[...], lmb1[...], lm2[...], lmb2[...])
            nm_o[...] = _mlp(ln_new, lm1[...], lmb1[...], lm2[...], lmb2[...])
            cm_o[...] = _mlp(c_new, cm1[...], cmb1[...], cm2[...], cmb2[...])

    w = _wspec((HID, HID))
    b = _wspec((1, HID))
    n_out = 6 if with_msgs else 3
    wspecs = [w, w, b, w, b, w, w, w, b, w, b]
    if with_msgs:
        wspecs += [w, b, w, b, w, b, w, b]
    return pl.pallas_call(
        body,
        grid=(GRID,),
        in_specs=[_bspec()] * 6 + wspecs,
        out_specs=[_bspec()] * n_out,
        out_shape=[jax.ShapeDtypeStruct((NHALF, HID), jnp.float32)] * n_out,
    )(l2c, pc2l, nc2l, lp, ln, c, *wd)


def kernel(l_embedding, c_embedding, pos_edge_index, neg_edge_index,
           lm_W1, lm_b1, lm_W2, lm_b2,
           cm_W1, cm_b1, cm_W2, cm_b2,
           lu_W1, lu_b1, lu_W2, lu_b2,
           cu_W1, cu_b1, cu_W2, cu_b2):
    lp = l_embedding[:NHALF]
    ln = l_embedding[NHALF:]
    c = c_embedding

    def glay(x):
        return x.reshape(NSUB, EPT).astype(jnp.int32)

    def slay(x):
        return x.reshape(NSUB, NCHUNK, 1, KCH).astype(jnp.int32)

    idx_arrays = []
    for arr in (pos_edge_index[0], pos_edge_index[1],
                neg_edge_index[0], neg_edge_index[1]):
        idx_arrays += [glay(arr), slay(arr)]
    zrows = jnp.zeros((ZROWS, HID), jnp.float32)

    lmb1 = lm_b1.reshape(1, HID)
    lmb2 = lm_b2.reshape(1, HID)
    cmb1 = cm_b1.reshape(1, HID)
    cmb2 = cm_b2.reshape(1, HID)
    lub1 = lu_b1.reshape(1, HID)
    lub2 = lu_b2.reshape(1, HID)
    cub1 = cu_b1.reshape(1, HID)
    cub2 = cu_b2.reshape(1, HID)
    cu1a, cu1b = cu_W1[:HID], cu_W1[HID:]
    lu1a, lu1b, lu1c = lu_W1[:HID], lu_W1[HID:2 * HID], lu_W1[2 * HID:]

    upd_w = (cu1a, cu1b, cub1, cu_W2, cub2,
             lu1a, lu1b, lu1c, lub1, lu_W2, lub2)
    msg_w = (lm_W1, lmb1, lm_W2, lmb2, cm_W1, cmb1, cm_W2, cmb2)

    pm, nm, cmsg = _tc_msgs(lp, ln, c, *msg_w)
    for r in range(NROUND):
        l2c, pc2l, nc2l = _sc_edge_sums(pm, nm, cmsg, idx_arrays, zrows)
        if r < NROUND - 1:
            lp, ln, c, pm, nm, cmsg = _tc_update(
                l2c, pc2l, nc2l, lp, ln, c, upd_w + msg_w, True)
        else:
            lp, ln, c = _tc_update(
                l2c, pc2l, nc2l, lp, ln, c, upd_w, False)

    return (jnp.concatenate([lp, ln], axis=0), c)
